# Initial kernel scaffold; baseline (speedup 1.0000x reference)
import jax
import jax.numpy as jnp
from jax.experimental import pallas as pl


def kernel(ze, codebook):
    raise NotImplementedError("write your pallas kernel here")


# speedup vs baseline: 2.1415x; 2.1415x over previous
import functools

import jax
import jax.numpy as jnp
from jax.experimental import pallas as pl
from jax.experimental.pallas import tpu as pltpu


def _vq_body(ze_ref, cb_ref, zq_ref, tok_ref, commit_ref, perp_ref,
             counts_ref, acc_ref, *, n_t_blocks, total_n, total_elems):
    b = pl.program_id(0)
    t = pl.program_id(1)
    nb = pl.num_programs(0)

    @pl.when(jnp.logical_and(b == 0, t == 0))
    def _init():
        counts_ref[...] = jnp.zeros_like(counts_ref)
        acc_ref[0] = 0.0

    zeb = ze_ref[0]
    cb = cb_ref[...]
    K = cb.shape[0]
    T = zeb.shape[1]

    scores = jax.lax.dot_general(
        cb, zeb, (((1,), (0,)), ((), ())), preferred_element_type=jnp.float32)
    cnorm = jnp.sum(cb * cb, axis=1, keepdims=True)
    zsq = jnp.sum(zeb * zeb, axis=0, keepdims=True)
    dists = (zsq + cnorm) - 2.0 * scores

    mind = jnp.min(dists, axis=0)
    kiota = jax.lax.broadcasted_iota(jnp.int32, (K, T), 0)
    idx = jnp.min(jnp.where(dists == mind[None, :], kiota, K), axis=0)

    tok_ref[0, 0, :] = idx

    onehot = jnp.where(kiota == idx[None, :], 1.0, 0.0)
    zqb = jax.lax.dot_general(
        cb, onehot, (((0,), (0,)), ((), ())), preferred_element_type=jnp.float32)
    zq_ref[0] = zqb

    ones_row = jnp.ones((1, T), dtype=jnp.float32)
    counts_ref[...] += jax.lax.dot_general(
        ones_row, onehot, (((1,), (1,)), ((), ())),
        preferred_element_type=jnp.float32)

    acc_ref[0] += jnp.sum(mind)

    @pl.when(jnp.logical_and(b == nb - 1, t == n_t_blocks - 1))
    def _fini():
        p = counts_ref[...] / total_n
        ent = jnp.sum(p * jnp.log(p + 1e-10))
        perp_ref[...] = jnp.exp(-ent).reshape(1, 1)
        commit_ref[...] = (0.25 * acc_ref[0] / total_elems).reshape(1, 1)


def kernel(ze, codebook):
    B, D, Tp = ze.shape
    K = codebook.shape[0]
    T = 512
    n_t_blocks = Tp // T
    grid = (B, n_t_blocks)

    body = functools.partial(
        _vq_body, n_t_blocks=n_t_blocks, total_n=float(B * Tp),
        total_elems=float(B * D * Tp))

    zq, tok, commit, perp = pl.pallas_call(
        body,
        grid=grid,
        in_specs=[
            pl.BlockSpec((1, D, T), lambda b, t: (b, 0, t)),
            pl.BlockSpec((K, D), lambda b, t: (0, 0)),
        ],
        out_specs=[
            pl.BlockSpec((1, D, T), lambda b, t: (b, 0, t)),
            pl.BlockSpec((1, 1, T), lambda b, t: (b, 0, t)),
            pl.BlockSpec((1, 1), lambda b, t: (0, 0)),
            pl.BlockSpec((1, 1), lambda b, t: (0, 0)),
        ],
        out_shape=[
            jax.ShapeDtypeStruct((B, D, Tp), jnp.float32),
            jax.ShapeDtypeStruct((B, 1, Tp), jnp.int32),
            jax.ShapeDtypeStruct((1, 1), jnp.float32),
            jax.ShapeDtypeStruct((1, 1), jnp.float32),
        ],
        scratch_shapes=[
            pltpu.VMEM((1, K), jnp.float32),
            pltpu.SMEM((1,), jnp.float32),
        ],
    )(ze, codebook)

    return (zq, tok.reshape(B, Tp), commit[0, 0], perp[0, 0])


# speedup vs baseline: 3.0123x; 1.4066x over previous
import functools

import jax
import jax.numpy as jnp
from jax.experimental import pallas as pl
from jax.experimental.pallas import tpu as pltpu


def _vq_body(ze_ref, cb_ref, zq_ref, tok_ref, commit_ref, perp_ref,
             counts_ref, acc_ref, *, n_t_blocks, total_n, total_elems):
    b = pl.program_id(0)
    t = pl.program_id(1)
    nb = pl.num_programs(0)

    @pl.when(jnp.logical_and(b == 0, t == 0))
    def _init():
        counts_ref[...] = jnp.zeros_like(counts_ref)
        acc_ref[0] = 0.0

    zeb = ze_ref[0]
    cb = cb_ref[...]
    K = cb.shape[0]
    T = zeb.shape[1]

    scores = jax.lax.dot_general(
        cb, zeb, (((1,), (0,)), ((), ())), preferred_element_type=jnp.float32)
    cnorm = jnp.sum(cb * cb, axis=1, keepdims=True)
    zsq = jnp.sum(zeb * zeb, axis=0, keepdims=True)
    dists = (zsq + cnorm) - 2.0 * scores

    mind = jnp.min(dists, axis=0)
    idx = jnp.argmin(dists, axis=0).astype(jnp.int32)
    kiota = jax.lax.broadcasted_iota(jnp.int32, (K, T), 0)

    tok_ref[0, 0, :] = idx

    onehot = jnp.where(kiota == idx[None, :], 1.0, 0.0)
    zqb = jax.lax.dot_general(
        cb, onehot, (((0,), (0,)), ((), ())), preferred_element_type=jnp.float32)
    zq_ref[0] = zqb

    ones_row = jnp.ones((1, T), dtype=jnp.float32)
    counts_ref[...] += jax.lax.dot_general(
        ones_row, onehot, (((1,), (1,)), ((), ())),
        preferred_element_type=jnp.float32)

    acc_ref[0] += jnp.sum(mind)

    @pl.when(jnp.logical_and(b == nb - 1, t == n_t_blocks - 1))
    def _fini():
        p = counts_ref[...] / total_n
        ent = jnp.sum(p * jnp.log(p + 1e-10))
        perp_ref[...] = jnp.exp(-ent).reshape(1, 1)
        commit_ref[...] = (0.25 * acc_ref[0] / total_elems).reshape(1, 1)


def kernel(ze, codebook):
    B, D, Tp = ze.shape
    K = codebook.shape[0]
    T = 4096
    n_t_blocks = Tp // T
    grid = (B, n_t_blocks)

    body = functools.partial(
        _vq_body, n_t_blocks=n_t_blocks, total_n=float(B * Tp),
        total_elems=float(B * D * Tp))

    zq, tok, commit, perp = pl.pallas_call(
        body,
        grid=grid,
        in_specs=[
            pl.BlockSpec((1, D, T), lambda b, t: (b, 0, t)),
            pl.BlockSpec((K, D), lambda b, t: (0, 0)),
        ],
        out_specs=[
            pl.BlockSpec((1, D, T), lambda b, t: (b, 0, t)),
            pl.BlockSpec((1, 1, T), lambda b, t: (b, 0, t)),
            pl.BlockSpec((1, 1), lambda b, t: (0, 0)),
            pl.BlockSpec((1, 1), lambda b, t: (0, 0)),
        ],
        out_shape=[
            jax.ShapeDtypeStruct((B, D, Tp), jnp.float32),
            jax.ShapeDtypeStruct((B, 1, Tp), jnp.int32),
            jax.ShapeDtypeStruct((1, 1), jnp.float32),
            jax.ShapeDtypeStruct((1, 1), jnp.float32),
        ],
        scratch_shapes=[
            pltpu.VMEM((1, K), jnp.float32),
            pltpu.SMEM((1,), jnp.float32),
        ],
    )(ze, codebook)

    return (zq, tok.reshape(B, Tp), commit[0, 0], perp[0, 0])
